# Initial kernel scaffold; baseline (speedup 1.0000x reference)
import jax
import jax.numpy as jnp
from jax.experimental import pallas as pl


def kernel(x, emb):
    raise NotImplementedError("write your pallas kernel here")


# speedup vs baseline: 1.6294x; 1.6294x over previous
import jax
import jax.numpy as jnp
from jax.experimental import pallas as pl


BLOCK_S = 512


def _add_kernel(x_ref, emb_ref, o_ref):
    o_ref[...] = x_ref[...] + emb_ref[...]


def kernel(x, emb):
    batch, seq, d = x.shape
    grid = (batch, seq // BLOCK_S)
    return pl.pallas_call(
        _add_kernel,
        grid=grid,
        in_specs=[
            pl.BlockSpec((1, BLOCK_S, d), lambda b, s: (b, s, 0)),
            pl.BlockSpec((BLOCK_S, d), lambda b, s: (s, 0)),
        ],
        out_specs=pl.BlockSpec((1, BLOCK_S, d), lambda b, s: (b, s, 0)),
        out_shape=jax.ShapeDtypeStruct((batch, seq, d), x.dtype),
    )(x, emb)


# speedup vs baseline: 1.9257x; 1.1818x over previous
import jax
import jax.numpy as jnp
from jax.experimental import pallas as pl


BLOCK_S = 512


def _add_kernel(x_ref, emb_ref, o_ref):
    o_ref[...] = x_ref[...] + emb_ref[...]


def kernel(x, emb):
    batch, seq, d = x.shape
    grid = (seq // BLOCK_S, batch)
    return pl.pallas_call(
        _add_kernel,
        grid=grid,
        in_specs=[
            pl.BlockSpec((1, BLOCK_S, d), lambda s, b: (b, s, 0)),
            pl.BlockSpec((BLOCK_S, d), lambda s, b: (s, 0)),
        ],
        out_specs=pl.BlockSpec((1, BLOCK_S, d), lambda s, b: (b, s, 0)),
        out_shape=jax.ShapeDtypeStruct((batch, seq, d), x.dtype),
    )(x, emb)
